# baseline (speedup 1.0000x reference)
import functools
import math

import jax
import jax.numpy as jnp
from jax import lax
from jax.experimental import pallas as pl
from jax.experimental.pallas import tpu as pltpu
from jax.experimental.pallas import tpu_sc as plsc

NC = 2
NS = 16
LN = 16
EB = 128
PB = 128


def _round_up(x, m):
    return (x + m - 1) // m * m


def _mesh():
    return plsc.VectorSubcoreMesh(core_axis_name="c", subcore_axis_name="s",
                                  num_cores=NC, num_subcores=NS)


def _tc_linear4(xp, wcat, bcat, oc, relu, np_):
    din = xp.shape[1]
    BN = 1024
    assert np_ % BN == 0
    nxb = -(-xp.shape[0] // BN)

    def body(x_ref, w_ref, b_ref, a_ref, b2_ref):
        xb = x_ref[...]
        if relu:
            xb = jnp.maximum(xb, 0.0)
        y = jnp.dot(xb, w_ref[...], preferred_element_type=jnp.float32)
        y = y + b_ref[...]
        a_ref[...] = y[:, :oc]
        b2_ref[...] = y[:, oc:]

    ospec = pl.BlockSpec((BN, oc), lambda i: (i, 0))
    call = pl.pallas_call(
        body,
        grid=(np_ // BN,),
        in_specs=[pl.BlockSpec((BN, din),
                               lambda i: (jnp.minimum(i, nxb - 1), 0)),
                  pl.BlockSpec((din, 2 * oc), lambda i: (0, 0)),
                  pl.BlockSpec((1, 2 * oc), lambda i: (0, 0))],
        out_specs=[ospec, ospec],
        out_shape=[jax.ShapeDtypeStruct((np_, oc), jnp.float32)] * 2,
    )
    q, k = call(xp, wcat[:, :2 * oc], bcat[:, :2 * oc])
    v, s = call(xp, wcat[:, 2 * oc:], bcat[:, 2 * oc:])
    return q, k, v, s


def _sc_edge_logits(q, k, srcp, dstp, invs):
    NP, oc = q.shape
    epad = srcp.shape[0]
    ew = epad // (NC * NS)
    nblk = ew // EB
    npair = nblk // 2
    nseg = oc // LN
    zspan = NP // NS
    zchunk = zspan // 4
    assert zspan % 4 == 0 and zchunk % LN == 0
    assert nblk % 2 == 0 and nblk >= 4

    @functools.partial(
        pl.kernel, mesh=_mesh(),
        compiler_params=pltpu.CompilerParams(needs_layout_passes=False, use_tc_tiling_on_sc=False),
        out_type=[jax.ShapeDtypeStruct((epad,), jnp.float32),
                  jax.ShapeDtypeStruct((NC, NP), jnp.float32)],
        scratch_types=[
            pltpu.VMEM((EB,), jnp.int32),
            pltpu.VMEM((EB,), jnp.int32),
            pltpu.VMEM((EB,), jnp.int32),
            pltpu.VMEM((EB,), jnp.int32),
            pltpu.VMEM((EB, oc), jnp.float32),
            pltpu.VMEM((EB, oc), jnp.float32),
            pltpu.VMEM((EB, oc), jnp.float32),
            pltpu.VMEM((EB, oc), jnp.float32),
            pltpu.VMEM((EB,), jnp.float32),
            pltpu.VMEM((EB,), jnp.float32),
            pltpu.VMEM((zchunk,), jnp.float32),
            pltpu.VMEM_SHARED((NP,), jnp.float32),
            pltpu.SemaphoreType.DMA,
            pltpu.SemaphoreType.DMA,
            pltpu.SemaphoreType.DMA,
            pltpu.SemaphoreType.DMA,
            pltpu.SemaphoreType.DMA,
            pltpu.SemaphoreType.DMA,
            pltpu.SemaphoreType.DMA,
            pltpu.SemaphoreType.DMA,
            pltpu.SemaphoreType.DMA,
            pltpu.SemaphoreType.DMA,
        ],
    )
    def kern(q_hbm, k_hbm, src_hbm, dst_hbm, p_hbm, asum_hbm,
             dstA, dstB, srcA, srcB, qA, qB, kA, kB, pA, pB, zbuf, asum_sp,
             semd0, semd1, sems0, sems1, semq0, semq1, semk0, semk1,
             semp0, semp1):
        cid = lax.axis_index("c")
        sid = lax.axis_index("s")
        wid = sid * NC + cid
        zv = jnp.zeros((LN,), jnp.float32)
        iota16 = lax.iota(jnp.int32, LN)
        sets = ((dstA, srcA, qA, kA, pA, semd0, sems0, semq0, semk0, semp0),
                (dstB, srcB, qB, kB, pB, semd1, sems1, semq1, semk1, semp1))

        def zfill(i, _):
            zbuf[pl.ds(i * LN, LN)] = zv
            return 0
        lax.fori_loop(0, zchunk // LN, zfill, 0)

        def zcopy(i, _):
            pltpu.sync_copy(zbuf, asum_sp.at[pl.ds(sid * zspan + i * zchunk,
                                                   zchunk)])
            return 0
        lax.fori_loop(0, zspan // zchunk, zcopy, 0)
        plsc.subcore_barrier()

        ebase = wid * ew

        def issue_idx(b, st):
            dv, sv = st[0], st[1]
            off = ebase + b * EB
            pltpu.async_copy(dst_hbm.at[pl.ds(off, EB)], dv, st[5])
            pltpu.async_copy(src_hbm.at[pl.ds(off, EB)], sv, st[6])

        def wait_idx(st):
            pltpu.make_async_copy(dst_hbm.at[pl.ds(0, EB)], st[0], st[5]).wait()
            pltpu.make_async_copy(src_hbm.at[pl.ds(0, EB)], st[1], st[6]).wait()

        def issue_gather(st):
            pltpu.async_copy(q_hbm.at[st[0]], st[2], st[7])
            pltpu.async_copy(k_hbm.at[st[1]], st[3], st[8])

        def wait_gather(st):
            pltpu.make_async_copy(q_hbm.at[st[0]], st[2], st[7]).wait()
            pltpu.make_async_copy(k_hbm.at[st[1]], st[3], st[8]).wait()

        def wait_pwrite(st):
            pltpu.make_async_copy(st[4], p_hbm.at[pl.ds(0, EB)], st[9]).wait()

        def body(b, st, other):
            @pl.when(b + 1 < nblk)
            def _():
                wait_idx(other)
                issue_gather(other)
            wait_gather(st)
            pl.when(b >= 2)(lambda: wait_pwrite(st))
            q_rows, k_rows, p_v = st[2], st[3], st[4]
            for g in range(EB // LN):
                alphav = jnp.zeros((LN,), jnp.float32)
                for el in range(LN):
                    i = g * LN + el
                    acc = jnp.zeros((LN,), jnp.float32)
                    for c in range(nseg):
                        sl = pl.ds(c * LN, LN)
                        acc = acc + q_rows[i, sl] * k_rows[i, sl]
                    alphav = jnp.where(iota16 == el, jnp.sum(acc), alphav)
                p_v[pl.ds(g * LN, LN)] = jnp.exp(alphav * invs)
            off = ebase + b * EB
            pltpu.async_copy(p_v, p_hbm.at[pl.ds(off, EB)], st[9])
            pltpu.sync_copy(p_v, asum_sp.at[st[0]], add=True)

            @pl.when(b + 2 < nblk)
            def _():
                issue_idx(b + 2, st)

        issue_idx(0, sets[0])
        wait_idx(sets[0])
        issue_gather(sets[0])
        issue_idx(1, sets[1])

        def pair(i, _):
            body(2 * i, sets[0], sets[1])
            body(2 * i + 1, sets[1], sets[0])
            return 0
        lax.fori_loop(0, npair, pair, 0)
        wait_pwrite(sets[0])
        wait_pwrite(sets[1])
        plsc.subcore_barrier()

        def acopy(i, _):
            sl = pl.ds(sid * zspan + i * zchunk, zchunk)
            pltpu.sync_copy(asum_sp.at[sl], asum_hbm.at[cid, sl])
            return 0
        lax.fori_loop(0, zspan // zchunk, acopy, 0)

    return kern(q, k, srcp, dstp)


def _sc_scatter_out(v, s, p, srcp, dstp, asum, cs, nch, piece, nout):
    NP, oc = v.shape
    epad = srcp.shape[0]
    ew = epad // NS
    EBC = 704
    nblk = ew // EBC
    npair = nblk // 2
    ngrp = EBC // LN
    nseg = oc // LN
    tr = cs // NS
    npieces = tr // piece
    cpc = nch // NC
    assert ew % EBC == 0 and nblk % 2 == 0 and tr % piece == 0

    @functools.partial(
        pl.kernel, mesh=_mesh(),
        compiler_params=pltpu.CompilerParams(needs_layout_passes=False, use_tc_tiling_on_sc=False),
        out_type=jax.ShapeDtypeStruct((nout, oc), jnp.float32),
        scratch_types=[
            pltpu.VMEM((EBC,), jnp.int32),
            pltpu.VMEM((EBC,), jnp.int32),
            pltpu.VMEM((EBC,), jnp.int32),
            pltpu.VMEM((EBC,), jnp.int32),
            pltpu.VMEM((EBC,), jnp.float32),
            pltpu.VMEM((EBC,), jnp.float32),
            pltpu.VMEM((PB + LN,), jnp.int32),
            pltpu.VMEM((PB + LN,), jnp.int32),
            pltpu.VMEM((PB + LN,), jnp.float32),
            pltpu.VMEM((PB,), jnp.int32),
            pltpu.VMEM((PB,), jnp.int32),
            pltpu.VMEM((PB,), jnp.float32),
            pltpu.VMEM((PB, oc), jnp.float32),
            pltpu.VMEM((piece, oc), jnp.float32),
            pltpu.VMEM((piece, oc), jnp.float32),
            pltpu.VMEM((piece,), jnp.float32),
            pltpu.VMEM((piece,), jnp.float32),
            pltpu.VMEM((piece,), jnp.float32),
            pltpu.VMEM_SHARED((cs + 8, oc), jnp.float32),
            pltpu.SemaphoreType.DMA,
            pltpu.SemaphoreType.DMA,
            pltpu.SemaphoreType.DMA,
            pltpu.SemaphoreType.DMA,
            pltpu.SemaphoreType.DMA,
            pltpu.SemaphoreType.DMA,
            pltpu.SemaphoreType.DMA,
        ],
    )
    def kern(v_hbm, s_hbm, p_hbm, src_hbm, dst_hbm, asum_hbm, out_hbm,
             dst0, dst1, src0, src1, p0, p1, stag_loc, stag_src, stag_p,
             fire_loc, fire_src, fire_p, rows_v, acc_v, s_v,
             a0_v, a1_v, scale_v, acc_sp,
             semd0, semd1, sems0, sems1, semp0, semp1, semg):
        cid = lax.axis_index("c")
        sid = lax.axis_index("s")
        zv = jnp.zeros((LN,), jnp.float32)
        zi16 = jnp.zeros((LN,), jnp.int32)
        bufs = ((dst0, src0, p0, semd0, sems0, semp0),
                (dst1, src1, p1, semd1, sems1, semp1))

        ebase = sid * ew
        iota16 = lax.iota(jnp.int32, LN)

        def issue(b, db, sb, pb, sd, ss, sp):
            off = ebase + b * EBC
            pltpu.async_copy(dst_hbm.at[pl.ds(off, EBC)], db, sd)
            pltpu.async_copy(src_hbm.at[pl.ds(off, EBC)], sb, ss)
            pltpu.async_copy(p_hbm.at[pl.ds(off, EBC)], pb, sp)

        def wait(db, sb, pb, sd, ss, sp):
            pltpu.make_async_copy(dst_hbm.at[pl.ds(0, EBC)], db, sd).wait()
            pltpu.make_async_copy(src_hbm.at[pl.ds(0, EBC)], sb, ss).wait()
            pltpu.make_async_copy(p_hbm.at[pl.ds(0, EBC)], pb, sp).wait()

        def fire():
            for t in range(PB // LN):
                sl = pl.ds(t * LN, LN)
                fire_loc[sl] = stag_loc[sl]
                fire_src[sl] = stag_src[sl]
                fire_p[sl] = stag_p[sl]
            pltpu.async_copy(v_hbm.at[fire_src], rows_v, semg).wait()

            def scale(i, _):
                pi = plsc.load_gather(fire_p, [zi16 + i])
                for c in range(nseg):
                    sl = pl.ds(c * LN, LN)
                    rows_v[i, sl] = rows_v[i, sl] * pi
                return 0
            lax.fori_loop(0, PB, scale, 0)
            pltpu.sync_copy(rows_v, acc_sp.at[fire_loc], add=True)
            lv = stag_loc[pl.ds(PB, LN)]
            sv = stag_src[pl.ds(PB, LN)]
            pv = stag_p[pl.ds(PB, LN)]
            stag_loc[pl.ds(0, LN)] = lv
            stag_src[pl.ds(0, LN)] = sv
            stag_p[pl.ds(0, LN)] = pv

        def chunk(ci, _):
            ch = ci * NC + cid
            lo = ch * cs
            hi = lo + cs

            def zfill(r, _):
                for c in range(nseg):
                    s_v[r, pl.ds(c * LN, LN)] = zv
                return 0
            lax.fori_loop(0, piece, zfill, 0)

            def zr(i, _):
                pltpu.sync_copy(s_v, acc_sp.at[pl.ds(sid * tr + i * piece,
                                                     piece)])
                return 0
            lax.fori_loop(0, npieces, zr, 0)
            plsc.subcore_barrier()

            issue(0, *bufs[0])
            issue(1, *bufs[1])

            def proc(b, db, sb, pb, sd, ss, sp, m):
                wait(db, sb, pb, sd, ss, sp)

                def grp(g, m):
                    sl = pl.ds(g * LN, LN)
                    d16 = db[sl]
                    msk = (d16 >= lo) & (d16 < hi)
                    plsc.store_compressed(stag_loc.at[pl.ds(m, LN)],
                                          d16 - lo, mask=msk)
                    plsc.store_compressed(stag_src.at[pl.ds(m, LN)],
                                          sb[sl], mask=msk)
                    plsc.store_compressed(stag_p.at[pl.ds(m, LN)],
                                          pb[sl], mask=msk)
                    m = m + jnp.sum(msk.astype(jnp.int32))
                    pl.when(m >= PB)(fire)
                    return jnp.where(m >= PB, m - PB, m)
                m = lax.fori_loop(0, ngrp, grp, m)

                @pl.when(b + 2 < nblk)
                def _():
                    issue(b + 2, db, sb, pb, sd, ss, sp)
                return m

            def pair(i, m):
                m = proc(2 * i, *bufs[0], m)
                m = proc(2 * i + 1, *bufs[1], m)
                return m
            m = lax.fori_loop(0, npair, pair, 0)

            for j in range((PB + LN) // LN):
                sl = pl.ds(j * LN, LN)
                pos = j * LN + iota16
                keep = pos < m
                stag_loc[sl] = jnp.where(keep, stag_loc[sl], cs)
                stag_src[sl] = jnp.where(keep, stag_src[sl], 0)
                stag_p[sl] = jnp.where(keep, stag_p[sl], 0.0)
            fire()
            plsc.subcore_barrier()

            def ep(i, _):
                lr = sid * tr + i * piece
                gr = lo + lr
                c0 = pltpu.async_copy(s_hbm.at[pl.ds(gr, piece)], s_v, semd0)
                c1 = pltpu.async_copy(asum_hbm.at[0, pl.ds(gr, piece)],
                                      a0_v, sems0)
                c2 = pltpu.async_copy(asum_hbm.at[1, pl.ds(gr, piece)],
                                      a1_v, semp0)
                pltpu.sync_copy(acc_sp.at[pl.ds(lr, piece)], acc_v)
                c0.wait()
                c1.wait()
                c2.wait()
                for jj in range(piece // LN):
                    sl = pl.ds(jj * LN, LN)
                    d16 = a0_v[sl] + a1_v[sl]
                    scale_v[sl] = jnp.where(d16 > 0.0, 1.0 / d16, 0.0)

                def row(r, _):
                    sc = plsc.load_gather(scale_v, [zi16 + r])
                    for c in range(nseg):
                        sl = pl.ds(c * LN, LN)
                        acc_v[r, sl] = s_v[r, sl] + acc_v[r, sl] * sc
                    return 0
                lax.fori_loop(0, piece, row, 0)

                @pl.when(gr + piece <= nout)
                def _():
                    pltpu.sync_copy(acc_v, out_hbm.at[pl.ds(gr, piece)])
                return 0
            lax.fori_loop(0, npieces, ep, 0)
            plsc.subcore_barrier()
            return 0
        lax.fori_loop(0, cpc, chunk, 0)

    return kern(v, s, p, srcp, dstp, asum)


def kernel(x, edge_index, weights, Wq1, bq1, Wk1, bk1, Wv1, bv1, Ws1, bs1,
           Wq2, bq2, Wk2, bk2, Wv2, bv2, Ws2, bs2):
    n, din = x.shape
    e = edge_index.shape[1]
    hc1 = Wq1.shape[1]
    oc2 = Wq2.shape[1]

    cs1 = 10240
    nch1 = 2 * (-(-n // (2 * cs1)))
    np_ = nch1 * cs1
    cs2 = 12800
    nch2 = np_ // cs2

    epad = _round_up(e, NC * NS * EB)
    src = edge_index[0]
    dst = edge_index[1]
    srcp = jnp.concatenate([src, jnp.zeros((epad - e,), jnp.int32)])
    dstp = jnp.concatenate([dst, jnp.full((epad - e,), n, jnp.int32)])

    w1 = jnp.concatenate([Wq1, Wk1, Wv1, Ws1], axis=1)
    b1 = jnp.concatenate([bq1, bk1, bv1, bs1]).reshape(1, -1)
    w2 = jnp.concatenate([Wq2, Wk2, Wv2, Ws2], axis=1)
    b2 = jnp.concatenate([bq2, bk2, bv2, bs2]).reshape(1, -1)

    q1, k1, v1, s1 = _tc_linear4(x, w1, b1, hc1, relu=False, np_=np_)
    p1, asum1 = _sc_edge_logits(q1, k1, srcp, dstp, 1.0 / math.sqrt(hc1))
    out1 = _sc_scatter_out(v1, s1, p1, srcp, dstp, asum1, cs1, nch1, 80, np_)

    q2, k2, v2, s2 = _tc_linear4(out1, w2, b2, oc2, relu=True, np_=np_)
    p2, asum2 = _sc_edge_logits(q2, k2, srcp, dstp, 1.0 / math.sqrt(oc2))
    out2 = _sc_scatter_out(v2, s2, p2, srcp, dstp, asum2, cs2, nch2, 160, n)
    return out2


# speedup vs baseline: 1.0600x; 1.0600x over previous
import functools
import math

import jax
import jax.numpy as jnp
from jax import lax
from jax.experimental import pallas as pl
from jax.experimental.pallas import tpu as pltpu
from jax.experimental.pallas import tpu_sc as plsc

NC = 2
NS = 16
LN = 16
EB = 128
PB = 128


def _round_up(x, m):
    return (x + m - 1) // m * m


def _mesh():
    return plsc.VectorSubcoreMesh(core_axis_name="c", subcore_axis_name="s",
                                  num_cores=NC, num_subcores=NS)


def _tc_linear4(xp, wcat, bcat, oc, relu, np_):
    din = xp.shape[1]
    BN = 1024
    assert np_ % BN == 0
    nxb = -(-xp.shape[0] // BN)

    def body(x_ref, w_ref, b_ref, q_ref, k_ref, v_ref, s_ref):
        xb = x_ref[...]
        if relu:
            xb = jnp.maximum(xb, 0.0)
        y = jnp.dot(xb, w_ref[...], preferred_element_type=jnp.float32)
        y = y + b_ref[...]
        q_ref[...] = y[:, :oc]
        k_ref[...] = y[:, oc:2 * oc]
        v_ref[...] = y[:, 2 * oc:3 * oc]
        s_ref[...] = y[:, 3 * oc:]

    ospec = pl.BlockSpec((BN, oc), lambda i: (i, 0))
    return pl.pallas_call(
        body,
        grid=(np_ // BN,),
        in_specs=[pl.BlockSpec((BN, din),
                               lambda i: (jnp.minimum(i, nxb - 1), 0)),
                  pl.BlockSpec((din, 4 * oc), lambda i: (0, 0)),
                  pl.BlockSpec((1, 4 * oc), lambda i: (0, 0))],
        out_specs=[ospec, ospec, ospec, ospec],
        out_shape=[jax.ShapeDtypeStruct((np_, oc), jnp.float32)] * 4,
    )(xp, wcat, bcat)


def _sc_edge_logits(q, k, srcp, dstp, invs):
    NP, oc = q.shape
    epad = srcp.shape[0]
    ew = epad // (NC * NS)
    nblk = ew // EB
    npair = nblk // 2
    nseg = oc // LN
    zspan = NP // NS
    zchunk = zspan // 4
    assert zspan % 4 == 0 and zchunk % LN == 0
    assert nblk % 2 == 0 and nblk >= 4

    @functools.partial(
        pl.kernel, mesh=_mesh(),
        compiler_params=pltpu.CompilerParams(needs_layout_passes=False, use_tc_tiling_on_sc=False),
        out_type=[jax.ShapeDtypeStruct((epad,), jnp.float32),
                  jax.ShapeDtypeStruct((NC, NP), jnp.float32)],
        scratch_types=[
            pltpu.VMEM((EB,), jnp.int32),
            pltpu.VMEM((EB,), jnp.int32),
            pltpu.VMEM((EB,), jnp.int32),
            pltpu.VMEM((EB,), jnp.int32),
            pltpu.VMEM((EB, oc), jnp.float32),
            pltpu.VMEM((EB, oc), jnp.float32),
            pltpu.VMEM((EB, oc), jnp.float32),
            pltpu.VMEM((EB, oc), jnp.float32),
            pltpu.VMEM((EB,), jnp.float32),
            pltpu.VMEM((EB,), jnp.float32),
            pltpu.VMEM((zchunk,), jnp.float32),
            pltpu.VMEM_SHARED((NP,), jnp.float32),
            pltpu.SemaphoreType.DMA,
            pltpu.SemaphoreType.DMA,
            pltpu.SemaphoreType.DMA,
            pltpu.SemaphoreType.DMA,
            pltpu.SemaphoreType.DMA,
            pltpu.SemaphoreType.DMA,
            pltpu.SemaphoreType.DMA,
            pltpu.SemaphoreType.DMA,
            pltpu.SemaphoreType.DMA,
            pltpu.SemaphoreType.DMA,
        ],
    )
    def kern(q_hbm, k_hbm, src_hbm, dst_hbm, p_hbm, asum_hbm,
             dstA, dstB, srcA, srcB, qA, qB, kA, kB, pA, pB, zbuf, asum_sp,
             semd0, semd1, sems0, sems1, semq0, semq1, semk0, semk1,
             semp0, semp1):
        cid = lax.axis_index("c")
        sid = lax.axis_index("s")
        wid = sid * NC + cid
        zv = jnp.zeros((LN,), jnp.float32)
        iota16 = lax.iota(jnp.int32, LN)
        sets = ((dstA, srcA, qA, kA, pA, semd0, sems0, semq0, semk0, semp0),
                (dstB, srcB, qB, kB, pB, semd1, sems1, semq1, semk1, semp1))

        def zfill(i, _):
            zbuf[pl.ds(i * LN, LN)] = zv
            return 0
        lax.fori_loop(0, zchunk // LN, zfill, 0)

        def zcopy(i, _):
            pltpu.sync_copy(zbuf, asum_sp.at[pl.ds(sid * zspan + i * zchunk,
                                                   zchunk)])
            return 0
        lax.fori_loop(0, zspan // zchunk, zcopy, 0)
        plsc.subcore_barrier()

        ebase = wid * ew

        def issue_idx(b, st):
            dv, sv = st[0], st[1]
            off = ebase + b * EB
            pltpu.async_copy(dst_hbm.at[pl.ds(off, EB)], dv, st[5])
            pltpu.async_copy(src_hbm.at[pl.ds(off, EB)], sv, st[6])

        def wait_idx(st):
            pltpu.make_async_copy(dst_hbm.at[pl.ds(0, EB)], st[0], st[5]).wait()
            pltpu.make_async_copy(src_hbm.at[pl.ds(0, EB)], st[1], st[6]).wait()

        def issue_gather(st):
            pltpu.async_copy(q_hbm.at[st[0]], st[2], st[7])
            pltpu.async_copy(k_hbm.at[st[1]], st[3], st[8])

        def wait_gather(st):
            pltpu.make_async_copy(q_hbm.at[st[0]], st[2], st[7]).wait()
            pltpu.make_async_copy(k_hbm.at[st[1]], st[3], st[8]).wait()

        def wait_pwrite(st):
            pltpu.make_async_copy(st[4], p_hbm.at[pl.ds(0, EB)], st[9]).wait()

        def body(b, st, other):
            @pl.when(b + 1 < nblk)
            def _():
                wait_idx(other)
                issue_gather(other)
            wait_gather(st)
            pl.when(b >= 2)(lambda: wait_pwrite(st))
            q_rows, k_rows, p_v = st[2], st[3], st[4]
            for g in range(EB // LN):
                alphav = jnp.zeros((LN,), jnp.float32)
                for el in range(LN):
                    i = g * LN + el
                    acc = jnp.zeros((LN,), jnp.float32)
                    for c in range(nseg):
                        sl = pl.ds(c * LN, LN)
                        acc = acc + q_rows[i, sl] * k_rows[i, sl]
                    alphav = jnp.where(iota16 == el, jnp.sum(acc), alphav)
                p_v[pl.ds(g * LN, LN)] = jnp.exp(alphav * invs)
            off = ebase + b * EB
            pltpu.async_copy(p_v, p_hbm.at[pl.ds(off, EB)], st[9])
            pltpu.sync_copy(p_v, asum_sp.at[st[0]], add=True)

            @pl.when(b + 2 < nblk)
            def _():
                issue_idx(b + 2, st)

        issue_idx(0, sets[0])
        wait_idx(sets[0])
        issue_gather(sets[0])
        issue_idx(1, sets[1])

        def pair(i, _):
            body(2 * i, sets[0], sets[1])
            body(2 * i + 1, sets[1], sets[0])
            return 0
        lax.fori_loop(0, npair, pair, 0)
        wait_pwrite(sets[0])
        wait_pwrite(sets[1])
        plsc.subcore_barrier()

        def acopy(i, _):
            sl = pl.ds(sid * zspan + i * zchunk, zchunk)
            pltpu.sync_copy(asum_sp.at[sl], asum_hbm.at[cid, sl])
            return 0
        lax.fori_loop(0, zspan // zchunk, acopy, 0)

    return kern(q, k, srcp, dstp)


def _sc_scatter_out(v, s, p, srcp, dstp, asum, cs, nch, piece, nout):
    NP, oc = v.shape
    epad = srcp.shape[0]
    ew = epad // NS
    EBC = 704
    nblk = ew // EBC
    npair = nblk // 2
    ngrp = EBC // LN
    nseg = oc // LN
    tr = cs // NS
    npieces = tr // piece
    cpc = nch // NC
    zrows_n = PB
    while tr % zrows_n:
        zrows_n -= 4
    assert ew % EBC == 0 and nblk % 2 == 0 and tr % piece == 0
    assert npieces % 2 == 0

    @functools.partial(
        pl.kernel, mesh=_mesh(),
        compiler_params=pltpu.CompilerParams(needs_layout_passes=False, use_tc_tiling_on_sc=False),
        out_type=jax.ShapeDtypeStruct((nout, oc), jnp.float32),
        scratch_types=[
            pltpu.VMEM((EBC,), jnp.int32),
            pltpu.VMEM((EBC,), jnp.int32),
            pltpu.VMEM((EBC,), jnp.int32),
            pltpu.VMEM((EBC,), jnp.int32),
            pltpu.VMEM((EBC,), jnp.float32),
            pltpu.VMEM((EBC,), jnp.float32),
            pltpu.VMEM((PB + LN,), jnp.int32),
            pltpu.VMEM((PB + LN,), jnp.int32),
            pltpu.VMEM((PB + LN,), jnp.float32),
            pltpu.VMEM((PB,), jnp.int32),
            pltpu.VMEM((PB,), jnp.int32),
            pltpu.VMEM((PB,), jnp.float32),
            pltpu.VMEM((PB, oc), jnp.float32),
            pltpu.VMEM((piece, oc), jnp.float32),
            pltpu.VMEM((piece, oc), jnp.float32),
            pltpu.VMEM((piece, oc), jnp.float32),
            pltpu.VMEM((piece, oc), jnp.float32),
            pltpu.VMEM((piece,), jnp.float32),
            pltpu.VMEM((piece,), jnp.float32),
            pltpu.VMEM((piece,), jnp.float32),
            pltpu.VMEM((piece,), jnp.float32),
            pltpu.VMEM((piece,), jnp.float32),
            pltpu.VMEM((piece,), jnp.float32),
            pltpu.VMEM_SHARED((cs + 8, oc), jnp.float32),
            pltpu.SemaphoreType.DMA,
            pltpu.SemaphoreType.DMA,
            pltpu.SemaphoreType.DMA,
            pltpu.SemaphoreType.DMA,
            pltpu.SemaphoreType.DMA,
            pltpu.SemaphoreType.DMA,
            pltpu.SemaphoreType.DMA,
            pltpu.SemaphoreType.DMA,
            pltpu.SemaphoreType.DMA,
            pltpu.SemaphoreType.DMA,
        ],
    )
    def kern(v_hbm, s_hbm, p_hbm, src_hbm, dst_hbm, asum_hbm, out_hbm,
             dst0, dst1, src0, src1, p0, p1, stag_loc, stag_src, stag_p,
             fire_loc, fire_src, fire_p, rows_v, acc_v0, acc_v1, s_v0, s_v1,
             a0_0, a0_1, a1_0, a1_1, sc_0, sc_1, acc_sp,
             semd0, semd1, sems0, sems1, semp0, semp1, semg,
             semo0, semo1, semz):
        cid = lax.axis_index("c")
        sid = lax.axis_index("s")
        zv = jnp.zeros((LN,), jnp.float32)
        zi16 = jnp.zeros((LN,), jnp.int32)
        bufs = ((dst0, src0, p0, semd0, sems0, semp0),
                (dst1, src1, p1, semd1, sems1, semp1))

        ebase = sid * ew
        iota16 = lax.iota(jnp.int32, LN)

        def issue(b, db, sb, pb, sd, ss, sp):
            off = ebase + b * EBC
            pltpu.async_copy(dst_hbm.at[pl.ds(off, EBC)], db, sd)
            pltpu.async_copy(src_hbm.at[pl.ds(off, EBC)], sb, ss)
            pltpu.async_copy(p_hbm.at[pl.ds(off, EBC)], pb, sp)

        def wait(db, sb, pb, sd, ss, sp):
            pltpu.make_async_copy(dst_hbm.at[pl.ds(0, EBC)], db, sd).wait()
            pltpu.make_async_copy(src_hbm.at[pl.ds(0, EBC)], sb, ss).wait()
            pltpu.make_async_copy(p_hbm.at[pl.ds(0, EBC)], pb, sp).wait()

        def fire():
            for t in range(PB // LN):
                sl = pl.ds(t * LN, LN)
                fire_loc[sl] = stag_loc[sl]
                fire_src[sl] = stag_src[sl]
                fire_p[sl] = stag_p[sl]
            pltpu.async_copy(v_hbm.at[fire_src], rows_v, semg).wait()

            def scale(i, _):
                pi = plsc.load_gather(fire_p, [zi16 + i])
                for c in range(nseg):
                    sl = pl.ds(c * LN, LN)
                    rows_v[i, sl] = rows_v[i, sl] * pi
                return 0
            lax.fori_loop(0, PB, scale, 0)
            pltpu.sync_copy(rows_v, acc_sp.at[fire_loc], add=True)
            lv = stag_loc[pl.ds(PB, LN)]
            sv = stag_src[pl.ds(PB, LN)]
            pv = stag_p[pl.ds(PB, LN)]
            stag_loc[pl.ds(0, LN)] = lv
            stag_src[pl.ds(0, LN)] = sv
            stag_p[pl.ds(0, LN)] = pv

        def chunk(ci, _):
            ch = ci * NC + cid
            lo = ch * cs
            hi = lo + cs

            def zfill(r, _):
                for c in range(nseg):
                    rows_v[r, pl.ds(c * LN, LN)] = zv
                return 0
            lax.fori_loop(0, zrows_n, zfill, 0)
            for i in range(tr // zrows_n):
                pltpu.async_copy(
                    rows_v.at[pl.ds(0, zrows_n)],
                    acc_sp.at[pl.ds(sid * tr + i * zrows_n, zrows_n)], semz)
            for i in range(tr // zrows_n):
                pltpu.make_async_copy(
                    rows_v.at[pl.ds(0, zrows_n)],
                    acc_sp.at[pl.ds(sid * tr, zrows_n)], semz).wait()
            plsc.subcore_barrier()

            issue(0, *bufs[0])
            issue(1, *bufs[1])

            def proc(b, db, sb, pb, sd, ss, sp, m):
                wait(db, sb, pb, sd, ss, sp)

                def grp(g, m):
                    sl = pl.ds(g * LN, LN)
                    d16 = db[sl]
                    msk = (d16 >= lo) & (d16 < hi)
                    plsc.store_compressed(stag_loc.at[pl.ds(m, LN)],
                                          d16 - lo, mask=msk)
                    plsc.store_compressed(stag_src.at[pl.ds(m, LN)],
                                          sb[sl], mask=msk)
                    plsc.store_compressed(stag_p.at[pl.ds(m, LN)],
                                          pb[sl], mask=msk)
                    m = m + jnp.sum(msk.astype(jnp.int32))
                    pl.when(m >= PB)(fire)
                    return jnp.where(m >= PB, m - PB, m)
                m = lax.fori_loop(0, ngrp, grp, m)

                @pl.when(b + 2 < nblk)
                def _():
                    issue(b + 2, db, sb, pb, sd, ss, sp)
                return m

            def pair(i, m):
                m = proc(2 * i, *bufs[0], m)
                m = proc(2 * i + 1, *bufs[1], m)
                return m
            m = lax.fori_loop(0, npair, pair, 0)

            for j in range((PB + LN) // LN):
                sl = pl.ds(j * LN, LN)
                pos = j * LN + iota16
                keep = pos < m
                stag_loc[sl] = jnp.where(keep, stag_loc[sl], cs)
                stag_src[sl] = jnp.where(keep, stag_src[sl], 0)
                stag_p[sl] = jnp.where(keep, stag_p[sl], 0.0)
            fire()
            plsc.subcore_barrier()

            esets = ((acc_v0, s_v0, a0_0, a1_0, sc_0, semd0, sems0, semp0,
                      semo0),
                     (acc_v1, s_v1, a0_1, a1_1, sc_1, semd1, sems1, semp1,
                      semo1))

            def valid(j):
                return lo + sid * tr + j * piece + piece <= nout

            def eissue(i, st):
                gr = lo + sid * tr + i * piece
                pltpu.async_copy(s_hbm.at[pl.ds(gr, piece)], st[1], st[5])
                pltpu.async_copy(asum_hbm.at[0, pl.ds(gr, piece)],
                                 st[2], st[6])
                pltpu.async_copy(asum_hbm.at[1, pl.ds(gr, piece)],
                                 st[3], st[7])

            def ewait(st):
                pltpu.make_async_copy(s_hbm.at[pl.ds(0, piece)],
                                      st[1], st[5]).wait()
                pltpu.make_async_copy(asum_hbm.at[0, pl.ds(0, piece)],
                                      st[2], st[6]).wait()
                pltpu.make_async_copy(asum_hbm.at[1, pl.ds(0, piece)],
                                      st[3], st[7]).wait()

            def ewaitout(st):
                pltpu.make_async_copy(st[0], out_hbm.at[pl.ds(0, piece)],
                                      st[8]).wait()

            def ebody(i, st):
                ewait(st)
                pl.when((i >= 2) & valid(i - 2))(lambda: ewaitout(st))
                lr = sid * tr + i * piece
                pltpu.sync_copy(acc_sp.at[pl.ds(lr, piece)], st[0])
                for jj in range(piece // LN):
                    sl = pl.ds(jj * LN, LN)
                    d16 = st[2][sl] + st[3][sl]
                    st[4][sl] = jnp.where(d16 > 0.0, 1.0 / d16, 0.0)

                def row(r, _):
                    sc = plsc.load_gather(st[4], [zi16 + r])
                    for c in range(nseg):
                        sl = pl.ds(c * LN, LN)
                        st[0][r, sl] = st[1][r, sl] + st[0][r, sl] * sc
                    return 0
                lax.fori_loop(0, piece, row, 0)
                gr = lo + lr

                @pl.when(valid(i))
                def _():
                    pltpu.async_copy(st[0], out_hbm.at[pl.ds(gr, piece)],
                                     st[8])

                @pl.when(i + 2 < npieces)
                def _():
                    eissue(i + 2, st)

            eissue(0, esets[0])
            eissue(1, esets[1])

            def epair(ii, _):
                ebody(2 * ii, esets[0])
                ebody(2 * ii + 1, esets[1])
                return 0
            lax.fori_loop(0, npieces // 2, epair, 0)
            pl.when(valid(npieces - 2))(lambda: ewaitout(esets[0]))
            pl.when(valid(npieces - 1))(lambda: ewaitout(esets[1]))
            plsc.subcore_barrier()
            return 0
        lax.fori_loop(0, cpc, chunk, 0)

    return kern(v, s, p, srcp, dstp, asum)


def kernel(x, edge_index, weights, Wq1, bq1, Wk1, bk1, Wv1, bv1, Ws1, bs1,
           Wq2, bq2, Wk2, bk2, Wv2, bv2, Ws2, bs2):
    n, din = x.shape
    e = edge_index.shape[1]
    hc1 = Wq1.shape[1]
    oc2 = Wq2.shape[1]

    cs1 = 10240
    nch1 = 2 * (-(-n // (2 * cs1)))
    np_ = nch1 * cs1
    cs2 = 12800
    nch2 = np_ // cs2

    epad = _round_up(e, NC * NS * EB)
    src = edge_index[0]
    dst = edge_index[1]
    srcp = jnp.concatenate([src, jnp.zeros((epad - e,), jnp.int32)])
    dstp = jnp.concatenate([dst, jnp.full((epad - e,), n, jnp.int32)])

    w1 = jnp.concatenate([Wq1, Wk1, Wv1, Ws1], axis=1)
    b1 = jnp.concatenate([bq1, bk1, bv1, bs1]).reshape(1, -1)
    w2 = jnp.concatenate([Wq2, Wk2, Wv2, Ws2], axis=1)
    b2 = jnp.concatenate([bq2, bk2, bv2, bs2]).reshape(1, -1)

    q1, k1, v1, s1 = _tc_linear4(x, w1, b1, hc1, relu=False, np_=np_)
    p1, asum1 = _sc_edge_logits(q1, k1, srcp, dstp, 1.0 / math.sqrt(hc1))
    out1 = _sc_scatter_out(v1, s1, p1, srcp, dstp, asum1, cs1, nch1, 40, np_)

    q2, k2, v2, s2 = _tc_linear4(out1, w2, b2, oc2, relu=True, np_=np_)
    p2, asum2 = _sc_edge_logits(q2, k2, srcp, dstp, 1.0 / math.sqrt(oc2))
    out2 = _sc_scatter_out(v2, s2, p2, srcp, dstp, asum2, cs2, nch2, 80, n)
    return out2
